# baseline (speedup 1.0000x reference)
import functools

import numpy as np
import jax
import jax.numpy as jnp
from jax import lax
from jax.experimental import pallas as pl
from jax.experimental.pallas import tpu as pltpu
from jax.experimental.pallas import tpu_sc as plsc

B = 512
P = 1039
C = 64
NQ = 37
G = NQ * NQ
NUSED = 1027
ROWPAD = 65
CH = 32
NTILES = 32
BPT = B // NTILES
PCH = (NUSED + 15) // 16


def _grid_positions():
    dirs = [(1, 0), (0, 1), (-1, 1), (-1, 0), (0, -1), (1, -1)]
    coords = [(0, 0)]
    k = 1
    while len(coords) < P:
        q, r = 0, -k
        for d in range(6):
            for _ in range(k):
                coords.append((q, r))
                q += dirs[d][0]
                r += dirs[d][1]
        k += 1
    coords = coords[:P]
    gpos = np.zeros((PCH * 16,), dtype=np.int32)
    for p, (q, r) in enumerate(coords):
        if abs(q) <= 18 and abs(r) <= 18 and p < NUSED:
            gpos[p] = (q + 18) * NQ + (r + 18)
    return gpos


_GPOS = _grid_positions()


def _sc_kernel(hexa_hbm, gpos_hbm, zeros_hbm, out_hbm, x_v, o_v, gpos_v):
    wid = lax.axis_index("s") * 2 + lax.axis_index("c")
    iota = lax.broadcasted_iota(jnp.int32, (16,), 0)

    pltpu.sync_copy(gpos_hbm, gpos_v)
    pltpu.sync_copy(zeros_hbm, o_v)

    b0 = wid * BPT

    def step(s, carry):
        b = b0 + s
        pltpu.sync_copy(
            hexa_hbm.at[b],
            x_v.at[pl.ds(0, P), pl.ds(0, C)],
        )

        for half in range(C // CH):
            def pchunk(pc, carry2):
                pbase = pc * 16
                rows = pbase + iota
                gp = gpos_v[pl.ds(pbase, 16)]
                msk = rows < NUSED
                for cc in range(CH):
                    cvec = jnp.full((16,), half * CH + cc, jnp.int32)
                    ovec = jnp.full((16,), cc, jnp.int32)
                    vals = plsc.load_gather(x_v, [rows, cvec])
                    plsc.store_scatter(o_v, [ovec, gp], vals, mask=msk)
                return carry2

            lax.fori_loop(0, PCH, pchunk, 0)
            pltpu.sync_copy(o_v, out_hbm.at[b, pl.ds(half * CH, CH), :])
        return carry

    lax.fori_loop(0, BPT, step, 0)


def kernel(hexa):
    mesh = plsc.VectorSubcoreMesh(core_axis_name="c", subcore_axis_name="s")
    run = functools.partial(
        pl.kernel,
        mesh=mesh,
        compiler_params=pltpu.CompilerParams(
            needs_layout_passes=False, use_tc_tiling_on_sc=False
        ),
        out_type=jax.ShapeDtypeStruct((B, C, G), jnp.float32),
        scratch_types=[
            pltpu.VMEM((PCH * 16 + 16, ROWPAD), jnp.float32),
            pltpu.VMEM((CH, G), jnp.float32),
            pltpu.VMEM((PCH * 16,), jnp.int32),
        ],
    )(_sc_kernel)
    gpos = jnp.asarray(_GPOS)
    zeros = jnp.zeros((CH, G), jnp.float32)
    out = run(hexa, gpos, zeros)
    return out.reshape(B, C, NQ, NQ)
